# baseline (speedup 1.0000x reference)
import jax
import jax.numpy as jnp
from jax import lax
from jax.experimental import pallas as pl
from jax.experimental.pallas import tpu as pltpu
from jax.experimental.pallas import tpu_sc as plsc

N_NODES = 10000
N_EDGES = 320000
D_NODE = 128
D_EDGE = 16
D_OUT = 128
D_HALF = D_NODE // 2

NS = 16
EDGES_PER_TILE = N_EDGES // NS
CHUNK = 80
N_CHUNKS = EDGES_PER_TILE // CHUNK
ROWS_PER_TILE = N_NODES // NS
ZROWS = 125
NSLOT = 10
GROUPS = N_CHUNKS // NSLOT


def _make_sc_body(use_ef):
  def _sc_body(*args):
    if use_ef:
      (x_hbm, src_hbm, dst_hbm, ef_hbm, a_out, ec_out), sc = args[:6], args[6:]
    else:
      (x_hbm, src_hbm, dst_hbm, a_out, ec_out), sc = args[:5], args[5:]
    src = sc[0:NSLOT]
    dst = sc[NSLOT:2 * NSLOT]
    rows = sc[2 * NSLOT:3 * NSLOT]
    efv = sc[3 * NSLOT:4 * NSLOT]
    ones_v = sc[4 * NSLOT]
    zbuf = sc[4 * NSLOT + 1]
    zbuf_e = sc[4 * NSLOT + 2]
    a_sh = sc[4 * NSLOT + 3]
    ec_sh = sc[4 * NSLOT + 4]
    sem_ld = sc[4 * NSLOT + 5:5 * NSLOT + 5]
    sem_g = sc[5 * NSLOT + 5:6 * NSLOT + 5]
    sem_s = sc[6 * NSLOT + 5:7 * NSLOT + 5]

    s = lax.axis_index("s")

    def _ones_row(i, carry):
      ones_v[i, :] = jnp.ones((16,), jnp.float32)
      return carry
    lax.fori_loop(0, CHUNK, _ones_row, 0)

    def _zero_row(i, carry):
      for j in range(D_HALF // 16):
        zbuf[i, pl.ds(j * 16, 16)] = jnp.zeros((16,), jnp.float32)
      zbuf_e[i, :] = jnp.zeros((16,), jnp.float32)
      return carry
    lax.fori_loop(0, ZROWS, _zero_row, 0)

    r0 = s * ROWS_PER_TILE
    for q in range(ROWS_PER_TILE // ZROWS):
      pltpu.sync_copy(zbuf, a_sh.at[pl.ds(r0 + q * ZROWS, ZROWS)])
      pltpu.sync_copy(zbuf_e, ec_sh.at[pl.ds(r0 + q * ZROWS, ZROWS)])
    plsc.subcore_barrier()

    tile_base = s * EDGES_PER_TILE

    def _issue_loads(k, base):
      pltpu.async_copy(src_hbm.at[pl.ds(base, CHUNK)], src[k], sem_ld[k])
      pltpu.async_copy(dst_hbm.at[pl.ds(base, CHUNK)], dst[k], sem_ld[k])
      if use_ef:
        pltpu.async_copy(ef_hbm.at[pl.ds(base, CHUNK), :], efv[k], sem_ld[k])

    def _wait_loads(k):
      pltpu.make_async_copy(src_hbm.at[pl.ds(0, CHUNK)], src[k],
                            sem_ld[k]).wait()
      pltpu.make_async_copy(dst_hbm.at[pl.ds(0, CHUNK)], dst[k],
                            sem_ld[k]).wait()
      if use_ef:
        pltpu.make_async_copy(ef_hbm.at[pl.ds(0, CHUNK), :], efv[k],
                              sem_ld[k]).wait()

    def _wait_gather(k):
      pltpu.make_async_copy(x_hbm.at[src[k]], rows[k], sem_g[k]).wait()

    ec_rows = efv if use_ef else [ones_v] * NSLOT

    def _wait_scat(k):
      pltpu.make_async_copy(rows[k], a_sh.at[dst[k]], sem_s[k]).wait()
      pltpu.make_async_copy(ec_rows[k], ec_sh.at[dst[k]], sem_s[k]).wait()

    for k in range(NSLOT):
      _issue_loads(k, tile_base + k * CHUNK)

    def _group(g, carry):
      base0 = tile_base + g * (NSLOT * CHUNK)
      for k in range(NSLOT):
        _wait_loads(k)
        pltpu.async_copy(x_hbm.at[src[k]], rows[k], sem_g[k])
      for k in range(NSLOT):
        _wait_gather(k)
        pltpu.async_copy(rows[k], a_sh.at[dst[k]], sem_s[k], add=True)
        pltpu.async_copy(ec_rows[k], ec_sh.at[dst[k]], sem_s[k], add=True)
      for k in range(NSLOT):
        _wait_scat(k)

        @pl.when(g < GROUPS - 1)
        def _():
          _issue_loads(k, base0 + (NSLOT + k) * CHUNK)
      return carry
    lax.fori_loop(0, GROUPS, _group, 0)

    plsc.subcore_barrier()

    @pl.when(s == 0)
    def _():
      pltpu.sync_copy(a_sh, a_out)

    @pl.when(s == 1)
    def _():
      pltpu.sync_copy(ec_sh, ec_out)

  return _sc_body


_SC_SCRATCH = (
    tuple(pltpu.VMEM((CHUNK,), jnp.int32) for _ in range(NSLOT))
    + tuple(pltpu.VMEM((CHUNK,), jnp.int32) for _ in range(NSLOT))
    + tuple(pltpu.VMEM((CHUNK, D_HALF), jnp.float32)
            for _ in range(NSLOT))
    + tuple(pltpu.VMEM((CHUNK, D_EDGE), jnp.float32)
            for _ in range(NSLOT))
    + (
        pltpu.VMEM((CHUNK, D_EDGE), jnp.float32),
        pltpu.VMEM((ZROWS, D_HALF), jnp.float32),
        pltpu.VMEM((ZROWS, D_EDGE), jnp.float32),
        pltpu.VMEM_SHARED((N_NODES, D_HALF), jnp.float32),
        pltpu.VMEM_SHARED((N_NODES, D_EDGE), jnp.float32),
    )
    + tuple(pltpu.SemaphoreType.DMA for _ in range(3 * NSLOT))
)

_OUT_TYPE = (
    jax.ShapeDtypeStruct((N_NODES, D_HALF), jnp.float32),
    jax.ShapeDtypeStruct((N_NODES, D_EDGE), jnp.float32),
)


def _make_sc_call(use_ef):
  return pl.kernel(
      _make_sc_body(use_ef),
      out_type=_OUT_TYPE,
      mesh=plsc.VectorSubcoreMesh(core_axis_name="c", subcore_axis_name="s",
                                  num_cores=1),
      compiler_params=pltpu.CompilerParams(use_tc_tiling_on_sc=False),
      scratch_types=_SC_SCRATCH,
  )


_sc_call_ef = _make_sc_call(True)
_sc_call_cnt = _make_sc_call(False)


BLK = 1000


def _tc_body(a0_ref, a1_ref, e_ref, c_ref, x_ref, ws_ref, wd_ref, we_ref,
             b_ref, o_ref):
  cnt = c_ref[:, 0:1]
  acc = jnp.dot(a0_ref[...], ws_ref[:D_HALF, :],
                preferred_element_type=jnp.float32)
  acc += jnp.dot(a1_ref[...], ws_ref[D_HALF:, :],
                 preferred_element_type=jnp.float32)
  acc += jnp.dot(e_ref[...], we_ref[...], preferred_element_type=jnp.float32)
  acc += cnt * (jnp.dot(x_ref[...], wd_ref[...],
                        preferred_element_type=jnp.float32) + b_ref[...])
  o_ref[...] = jnp.maximum(acc / jnp.maximum(cnt, 1.0), 0.0)


_tc_call = pl.pallas_call(
    _tc_body,
    grid=(N_NODES // BLK,),
    in_specs=[
        pl.BlockSpec((BLK, D_HALF), lambda i: (i, 0)),
        pl.BlockSpec((BLK, D_HALF), lambda i: (i, 0)),
        pl.BlockSpec((BLK, D_EDGE), lambda i: (i, 0)),
        pl.BlockSpec((BLK, D_EDGE), lambda i: (i, 0)),
        pl.BlockSpec((BLK, D_NODE), lambda i: (i, 0)),
        pl.BlockSpec((D_NODE, D_OUT), lambda i: (0, 0)),
        pl.BlockSpec((D_NODE, D_OUT), lambda i: (0, 0)),
        pl.BlockSpec((D_EDGE, D_OUT), lambda i: (0, 0)),
        pl.BlockSpec((1, D_OUT), lambda i: (0, 0)),
    ],
    out_specs=pl.BlockSpec((BLK, D_OUT), lambda i: (i, 0)),
    out_shape=jax.ShapeDtypeStruct((N_NODES, D_OUT), jnp.float32),
)


def kernel(node_feats, edge_feats, edge_index, W, b):
  ei = edge_index.astype(jnp.int32)
  src = ei[0]
  dst = ei[1]
  xh0 = node_feats[:, :D_HALF]
  xh1 = node_feats[:, D_HALF:]
  a0_p, e_p = _sc_call_ef(xh0, src, dst, edge_feats)
  a1_p, c_p = _sc_call_cnt(xh1, src, dst)
  ws_t = W[:, :D_NODE].T
  wd_t = W[:, D_NODE:2 * D_NODE].T
  we_t = W[:, 2 * D_NODE:].T
  return _tc_call(a0_p, a1_p, e_p, c_p, node_feats, ws_t, wd_t, we_t,
                  b.reshape(1, D_OUT))


# speedup vs baseline: 1.0138x; 1.0138x over previous
import jax
import jax.numpy as jnp
from jax import lax
from jax.experimental import pallas as pl
from jax.experimental.pallas import tpu as pltpu
from jax.experimental.pallas import tpu_sc as plsc

N_NODES = 10000
N_EDGES = 320000
D_NODE = 128
D_EDGE = 16
D_OUT = 128
D_HALF = D_NODE // 2

NC = 2
NS = 16
CHUNK = 128
N_ROWS = N_EDGES // CHUNK
ROWS_TILE = N_ROWS // NS
REM = N_ROWS - ROWS_TILE * NS
ZONES = N_NODES // NS
ZROWS = 125
NSLOT = 6
GROUPS = ROWS_TILE // NSLOT


def _sc_body(xh0_hbm, xh1_hbm, src_hbm, dst_hbm, ef_hbm,
             a_out, e_out, c_out, *sc):
  src = sc[0:NSLOT]
  dst = sc[NSLOT:2 * NSLOT]
  rows = sc[2 * NSLOT:3 * NSLOT]
  efv = sc[3 * NSLOT:4 * NSLOT]
  ones_v = sc[4 * NSLOT]
  zbuf = sc[4 * NSLOT + 1]
  zbuf_e = sc[4 * NSLOT + 2]
  a_sh = sc[4 * NSLOT + 3]
  ec_sh = sc[4 * NSLOT + 4]
  sem_ld = sc[4 * NSLOT + 5:5 * NSLOT + 5]
  sem_g = sc[5 * NSLOT + 5:6 * NSLOT + 5]
  sem_s = sc[6 * NSLOT + 5:7 * NSLOT + 5]

  c = lax.axis_index("c")
  s = lax.axis_index("s")

  def _ones_row(i, carry):
    ones_v[i, :] = jnp.ones((16,), jnp.float32)
    return carry
  lax.fori_loop(0, CHUNK, _ones_row, 0)

  def _zero_row(i, carry):
    for j in range(D_HALF // 16):
      zbuf[i, pl.ds(j * 16, 16)] = jnp.zeros((16,), jnp.float32)
    zbuf_e[i, :] = jnp.zeros((16,), jnp.float32)
    return carry
  lax.fori_loop(0, ZROWS, _zero_row, 0)

  r0 = s * ZONES
  for q in range(ZONES // ZROWS):
    pltpu.sync_copy(zbuf, a_sh.at[pl.ds(r0 + q * ZROWS, ZROWS)])
    pltpu.sync_copy(zbuf_e, ec_sh.at[pl.ds(r0 + q * ZROWS, ZROWS)])
  plsc.subcore_barrier()

  tile_row0 = s * ROWS_TILE

  def _run(x_hbm, use_ef):
    def _issue_loads(k, row):
      pltpu.async_copy(src_hbm.at[row], src[k], sem_ld[k])
      pltpu.async_copy(dst_hbm.at[row], dst[k], sem_ld[k])
      if use_ef:
        pltpu.async_copy(ef_hbm.at[pl.ds(row * CHUNK, CHUNK), :], efv[k],
                         sem_ld[k])

    def _wait_loads(k):
      pltpu.make_async_copy(src_hbm.at[0], src[k], sem_ld[k]).wait()
      pltpu.make_async_copy(dst_hbm.at[0], dst[k], sem_ld[k]).wait()
      if use_ef:
        pltpu.make_async_copy(ef_hbm.at[pl.ds(0, CHUNK), :], efv[k],
                              sem_ld[k]).wait()

    def _wait_gather(k):
      pltpu.make_async_copy(x_hbm.at[src[k]], rows[k], sem_g[k]).wait()

    ec_rows = efv if use_ef else [ones_v] * NSLOT

    def _wait_scat(k):
      pltpu.make_async_copy(rows[k], a_sh.at[dst[k]], sem_s[k]).wait()
      pltpu.make_async_copy(ec_rows[k], ec_sh.at[dst[k]], sem_s[k]).wait()

    for k in range(NSLOT):
      _issue_loads(k, tile_row0 + k)

    def _group(g, carry):
      row0 = tile_row0 + g * NSLOT
      for k in range(NSLOT):
        _wait_loads(k)
        pltpu.async_copy(x_hbm.at[src[k]], rows[k], sem_g[k])
      for k in range(NSLOT):
        _wait_gather(k)
        pltpu.async_copy(rows[k], a_sh.at[dst[k]], sem_s[k], add=True)
        pltpu.async_copy(ec_rows[k], ec_sh.at[dst[k]], sem_s[k], add=True)
      for k in range(NSLOT):
        _wait_scat(k)

        @pl.when(g < GROUPS - 1)
        def _():
          _issue_loads(k, row0 + NSLOT + k)
      return carry
    lax.fori_loop(0, GROUPS, _group, 0)

    @pl.when(s < REM)
    def _():
      row = NS * ROWS_TILE + s
      pltpu.sync_copy(src_hbm.at[row], src[0])
      pltpu.sync_copy(dst_hbm.at[row], dst[0])
      if use_ef:
        pltpu.sync_copy(ef_hbm.at[pl.ds(row * CHUNK, CHUNK), :], efv[0])
      pltpu.sync_copy(x_hbm.at[src[0]], rows[0])
      pltpu.sync_copy(rows[0], a_sh.at[dst[0]], add=True)
      pltpu.sync_copy(ec_rows[0], ec_sh.at[dst[0]], add=True)

  @pl.when(c == 0)
  def _():
    _run(xh0_hbm, use_ef=True)

  @pl.when(c == 1)
  def _():
    _run(xh1_hbm, use_ef=False)

  plsc.subcore_barrier()

  @pl.when(jnp.logical_and(c == 0, s == 0))
  def _():
    pltpu.sync_copy(a_sh, a_out.at[0])

  @pl.when(jnp.logical_and(c == 1, s == 0))
  def _():
    pltpu.sync_copy(a_sh, a_out.at[1])

  @pl.when(jnp.logical_and(c == 0, s == 1))
  def _():
    pltpu.sync_copy(ec_sh, e_out)

  @pl.when(jnp.logical_and(c == 1, s == 1))
  def _():
    pltpu.sync_copy(ec_sh, c_out)


_sc_call = pl.kernel(
    _sc_body,
    out_type=(
        jax.ShapeDtypeStruct((NC, N_NODES, D_HALF), jnp.float32),
        jax.ShapeDtypeStruct((N_NODES, D_EDGE), jnp.float32),
        jax.ShapeDtypeStruct((N_NODES, D_EDGE), jnp.float32),
    ),
    mesh=plsc.VectorSubcoreMesh(core_axis_name="c", subcore_axis_name="s"),
    compiler_params=pltpu.CompilerParams(use_tc_tiling_on_sc=False),
    scratch_types=(
        tuple(pltpu.VMEM((CHUNK,), jnp.int32) for _ in range(NSLOT))
        + tuple(pltpu.VMEM((CHUNK,), jnp.int32) for _ in range(NSLOT))
        + tuple(pltpu.VMEM((CHUNK, D_HALF), jnp.float32)
                for _ in range(NSLOT))
        + tuple(pltpu.VMEM((CHUNK, D_EDGE), jnp.float32)
                for _ in range(NSLOT))
        + (
            pltpu.VMEM((CHUNK, D_EDGE), jnp.float32),
            pltpu.VMEM((ZROWS, D_HALF), jnp.float32),
            pltpu.VMEM((ZROWS, D_EDGE), jnp.float32),
            pltpu.VMEM_SHARED((N_NODES, D_HALF), jnp.float32),
            pltpu.VMEM_SHARED((N_NODES, D_EDGE), jnp.float32),
        )
        + tuple(pltpu.SemaphoreType.DMA for _ in range(3 * NSLOT))
    ),
)


BLK = 1000


def _tc_body(a_ref, e_ref, c_ref, x_ref, ws_ref, wd_ref, we_ref, b_ref,
             o_ref):
  cnt = c_ref[:, 0:1]
  acc = jnp.dot(a_ref[0], ws_ref[:D_HALF, :],
                preferred_element_type=jnp.float32)
  acc += jnp.dot(a_ref[1], ws_ref[D_HALF:, :],
                 preferred_element_type=jnp.float32)
  acc += jnp.dot(e_ref[...], we_ref[...], preferred_element_type=jnp.float32)
  acc += cnt * (jnp.dot(x_ref[...], wd_ref[...],
                        preferred_element_type=jnp.float32) + b_ref[...])
  o_ref[...] = jnp.maximum(acc / jnp.maximum(cnt, 1.0), 0.0)


_tc_call = pl.pallas_call(
    _tc_body,
    grid=(N_NODES // BLK,),
    in_specs=[
        pl.BlockSpec((NC, BLK, D_HALF), lambda i: (0, i, 0)),
        pl.BlockSpec((BLK, D_EDGE), lambda i: (i, 0)),
        pl.BlockSpec((BLK, D_EDGE), lambda i: (i, 0)),
        pl.BlockSpec((BLK, D_NODE), lambda i: (i, 0)),
        pl.BlockSpec((D_NODE, D_OUT), lambda i: (0, 0)),
        pl.BlockSpec((D_NODE, D_OUT), lambda i: (0, 0)),
        pl.BlockSpec((D_EDGE, D_OUT), lambda i: (0, 0)),
        pl.BlockSpec((1, D_OUT), lambda i: (0, 0)),
    ],
    out_specs=pl.BlockSpec((BLK, D_OUT), lambda i: (i, 0)),
    out_shape=jax.ShapeDtypeStruct((N_NODES, D_OUT), jnp.float32),
)


def kernel(node_feats, edge_feats, edge_index, W, b):
  ei = edge_index.astype(jnp.int32)
  src2d = ei[0].reshape(N_ROWS, CHUNK)
  dst2d = ei[1].reshape(N_ROWS, CHUNK)
  xh0 = node_feats[:, :D_HALF]
  xh1 = node_feats[:, D_HALF:]
  a_p, e_p, c_p = _sc_call(xh0, xh1, src2d, dst2d, edge_feats)
  ws_t = W[:, :D_NODE].T
  wd_t = W[:, D_NODE:2 * D_NODE].T
  we_t = W[:, 2 * D_NODE:].T
  return _tc_call(a_p, e_p, c_p, node_feats, ws_t, wd_t, we_t,
                  b.reshape(1, D_OUT))


# speedup vs baseline: 1.0307x; 1.0167x over previous
import jax
import jax.numpy as jnp
from jax import lax
from jax.experimental import pallas as pl
from jax.experimental.pallas import tpu as pltpu
from jax.experimental.pallas import tpu_sc as plsc

N_NODES = 10000
N_EDGES = 320000
D_NODE = 128
D_EDGE = 16
D_OUT = 128
D_HALF = D_NODE // 2

NC = 2
NS = 16
EDGES_PER_TILE = N_EDGES // NS
CHUNK = 40
N_CHUNKS = EDGES_PER_TILE // CHUNK
ROWS_PER_TILE = N_NODES // NS
ZROWS = 125
NSLOT = 20
GROUPS = N_CHUNKS // NSLOT


def _sc_body(xh0_hbm, xh1_hbm, ei_hbm, ef_hbm,
             a_out, e_out, c_out, *sc):
  src = sc[0:NSLOT]
  dst = sc[NSLOT:2 * NSLOT]
  rows = sc[2 * NSLOT:3 * NSLOT]
  efv = sc[3 * NSLOT:4 * NSLOT]
  ones_v = sc[4 * NSLOT]
  zbuf = sc[4 * NSLOT + 1]
  zbuf_e = sc[4 * NSLOT + 2]
  a_sh = sc[4 * NSLOT + 3]
  ec_sh = sc[4 * NSLOT + 4]
  sem_ld = sc[4 * NSLOT + 5:5 * NSLOT + 5]
  sem_g = sem_ld
  sem_s = sem_ld

  c = lax.axis_index("c")
  s = lax.axis_index("s")

  def _ones_row(i, carry):
    ones_v[i, :] = jnp.ones((16,), jnp.float32)
    return carry
  lax.fori_loop(0, CHUNK, _ones_row, 0)

  def _zero_row(i, carry):
    for j in range(D_HALF // 16):
      zbuf[i, pl.ds(j * 16, 16)] = jnp.zeros((16,), jnp.float32)
    zbuf_e[i, :] = jnp.zeros((16,), jnp.float32)
    return carry
  lax.fori_loop(0, ZROWS, _zero_row, 0)

  r0 = s * ROWS_PER_TILE
  for q in range(ROWS_PER_TILE // ZROWS):
    pltpu.sync_copy(zbuf, a_sh.at[pl.ds(r0 + q * ZROWS, ZROWS)])
    pltpu.sync_copy(zbuf_e, ec_sh.at[pl.ds(r0 + q * ZROWS, ZROWS)])
  plsc.subcore_barrier()

  tile_base = s * EDGES_PER_TILE

  def _run(x_hbm, use_ef):
    def _issue_loads(k, base):
      pltpu.async_copy(ei_hbm.at[0, pl.ds(base, CHUNK)], src[k], sem_ld[k])
      pltpu.async_copy(ei_hbm.at[1, pl.ds(base, CHUNK)], dst[k], sem_ld[k])
      if use_ef:
        pltpu.async_copy(ef_hbm.at[pl.ds(base, CHUNK), :], efv[k], sem_ld[k])

    def _wait_loads(k):
      pltpu.make_async_copy(ei_hbm.at[0, pl.ds(0, CHUNK)], src[k],
                            sem_ld[k]).wait()
      pltpu.make_async_copy(ei_hbm.at[1, pl.ds(0, CHUNK)], dst[k],
                            sem_ld[k]).wait()
      if use_ef:
        pltpu.make_async_copy(ef_hbm.at[pl.ds(0, CHUNK), :], efv[k],
                              sem_ld[k]).wait()

    def _wait_gather(k):
      pltpu.make_async_copy(x_hbm.at[src[k]], rows[k], sem_g[k]).wait()

    ec_rows = efv if use_ef else [ones_v] * NSLOT

    def _wait_scat(k):
      pltpu.make_async_copy(rows[k], a_sh.at[dst[k]], sem_s[k]).wait()
      pltpu.make_async_copy(ec_rows[k], ec_sh.at[dst[k]], sem_s[k]).wait()

    for k in range(NSLOT):
      _issue_loads(k, tile_base + k * CHUNK)

    def _group(g, carry):
      base0 = tile_base + g * (NSLOT * CHUNK)
      for k in range(NSLOT):
        _wait_loads(k)
        pltpu.async_copy(x_hbm.at[src[k]], rows[k], sem_g[k])
      for k in range(NSLOT):
        _wait_gather(k)
        pltpu.async_copy(rows[k], a_sh.at[dst[k]], sem_s[k], add=True)
        pltpu.async_copy(ec_rows[k], ec_sh.at[dst[k]], sem_s[k], add=True)
      for k in range(NSLOT):
        _wait_scat(k)

        @pl.when(g < GROUPS - 1)
        def _():
          _issue_loads(k, base0 + (NSLOT + k) * CHUNK)
      return carry
    lax.fori_loop(0, GROUPS, _group, 0)

  @pl.when(c == 0)
  def _():
    _run(xh0_hbm, use_ef=True)

  @pl.when(c == 1)
  def _():
    _run(xh1_hbm, use_ef=False)

  plsc.subcore_barrier()

  @pl.when(jnp.logical_and(c == 0, s == 0))
  def _():
    pltpu.sync_copy(a_sh, a_out.at[0])

  @pl.when(jnp.logical_and(c == 1, s == 0))
  def _():
    pltpu.sync_copy(a_sh, a_out.at[1])

  @pl.when(jnp.logical_and(c == 0, s == 1))
  def _():
    pltpu.sync_copy(ec_sh, e_out)

  @pl.when(jnp.logical_and(c == 1, s == 1))
  def _():
    pltpu.sync_copy(ec_sh, c_out)


_sc_call = pl.kernel(
    _sc_body,
    out_type=(
        jax.ShapeDtypeStruct((NC, N_NODES, D_HALF), jnp.float32),
        jax.ShapeDtypeStruct((N_NODES, D_EDGE), jnp.float32),
        jax.ShapeDtypeStruct((N_NODES, D_EDGE), jnp.float32),
    ),
    mesh=plsc.VectorSubcoreMesh(core_axis_name="c", subcore_axis_name="s"),
    compiler_params=pltpu.CompilerParams(use_tc_tiling_on_sc=False),
    scratch_types=(
        tuple(pltpu.VMEM((CHUNK,), jnp.int32) for _ in range(NSLOT))
        + tuple(pltpu.VMEM((CHUNK,), jnp.int32) for _ in range(NSLOT))
        + tuple(pltpu.VMEM((CHUNK, D_HALF), jnp.float32)
                for _ in range(NSLOT))
        + tuple(pltpu.VMEM((CHUNK, D_EDGE), jnp.float32)
                for _ in range(NSLOT))
        + (
            pltpu.VMEM((CHUNK, D_EDGE), jnp.float32),
            pltpu.VMEM((ZROWS, D_HALF), jnp.float32),
            pltpu.VMEM((ZROWS, D_EDGE), jnp.float32),
            pltpu.VMEM_SHARED((N_NODES, D_HALF), jnp.float32),
            pltpu.VMEM_SHARED((N_NODES, D_EDGE), jnp.float32),
        )
        + tuple(pltpu.SemaphoreType.DMA for _ in range(NSLOT))
    ),
)


BLK = 1000


def _tc_body(a_ref, e_ref, c_ref, x_ref, ws_ref, wd_ref, we_ref, b_ref,
             o_ref):
  cnt = c_ref[:, 0:1]
  acc = jnp.dot(a_ref[0], ws_ref[:D_HALF, :],
                preferred_element_type=jnp.float32)
  acc += jnp.dot(a_ref[1], ws_ref[D_HALF:, :],
                 preferred_element_type=jnp.float32)
  acc += jnp.dot(e_ref[...], we_ref[...], preferred_element_type=jnp.float32)
  acc += cnt * (jnp.dot(x_ref[...], wd_ref[...],
                        preferred_element_type=jnp.float32) + b_ref[...])
  o_ref[...] = jnp.maximum(acc / jnp.maximum(cnt, 1.0), 0.0)


_tc_call = pl.pallas_call(
    _tc_body,
    grid=(N_NODES // BLK,),
    in_specs=[
        pl.BlockSpec((NC, BLK, D_HALF), lambda i: (0, i, 0)),
        pl.BlockSpec((BLK, D_EDGE), lambda i: (i, 0)),
        pl.BlockSpec((BLK, D_EDGE), lambda i: (i, 0)),
        pl.BlockSpec((BLK, D_NODE), lambda i: (i, 0)),
        pl.BlockSpec((D_NODE, D_OUT), lambda i: (0, 0)),
        pl.BlockSpec((D_NODE, D_OUT), lambda i: (0, 0)),
        pl.BlockSpec((D_EDGE, D_OUT), lambda i: (0, 0)),
        pl.BlockSpec((1, D_OUT), lambda i: (0, 0)),
    ],
    out_specs=pl.BlockSpec((BLK, D_OUT), lambda i: (i, 0)),
    out_shape=jax.ShapeDtypeStruct((N_NODES, D_OUT), jnp.float32),
)


def kernel(node_feats, edge_feats, edge_index, W, b):
  ei = edge_index.astype(jnp.int32)
  xh0 = node_feats[:, :D_HALF]
  xh1 = node_feats[:, D_HALF:]
  a_p, e_p, c_p = _sc_call(xh0, xh1, ei, edge_feats)
  ws_t = W[:, :D_NODE].T
  wd_t = W[:, D_NODE:2 * D_NODE].T
  we_t = W[:, 2 * D_NODE:].T
  return _tc_call(a_p, e_p, c_p, node_feats, ws_t, wd_t, we_t,
                  b.reshape(1, D_OUT))


# speedup vs baseline: 1.0828x; 1.0505x over previous
import jax
import jax.numpy as jnp
from jax import lax
from jax.experimental import pallas as pl
from jax.experimental.pallas import tpu as pltpu
from jax.experimental.pallas import tpu_sc as plsc

N_NODES = 10000
N_EDGES = 320000
D_NODE = 128
D_EDGE = 16
D_OUT = 128
D_HALF = D_NODE // 2

NC = 2
NS = 16
EDGES_PER_TILE = N_EDGES // NS
CHUNK = 80
N_CHUNKS = EDGES_PER_TILE // CHUNK
ROWS_PER_TILE = N_NODES // NS
ZROWS = 125
NSLOT = 10
GROUPS = N_CHUNKS // NSLOT


def _sc_body(xh0_hbm, xh1_hbm, ei_hbm, ef_hbm,
             a_out, e_out, c_out, *sc):
  src = sc[0:NSLOT]
  dst = sc[NSLOT:2 * NSLOT]
  rows = sc[2 * NSLOT:3 * NSLOT]
  efv = sc[3 * NSLOT:4 * NSLOT]
  ones_v = sc[4 * NSLOT]
  zbuf = sc[4 * NSLOT + 1]
  zbuf_e = sc[4 * NSLOT + 2]
  a_sh = sc[4 * NSLOT + 3]
  ec_sh = sc[4 * NSLOT + 4]
  sem_ld = sc[4 * NSLOT + 5:5 * NSLOT + 5]
  sem_g = sc[5 * NSLOT + 5:6 * NSLOT + 5]
  sem_s = sc[6 * NSLOT + 5:7 * NSLOT + 5]

  c = lax.axis_index("c")
  s = lax.axis_index("s")

  def _ones_row(i, carry):
    ones_v[i, :] = jnp.ones((16,), jnp.float32)
    return carry
  lax.fori_loop(0, CHUNK, _ones_row, 0)

  def _zero_row(i, carry):
    for j in range(D_HALF // 16):
      zbuf[i, pl.ds(j * 16, 16)] = jnp.zeros((16,), jnp.float32)
    zbuf_e[i, :] = jnp.zeros((16,), jnp.float32)
    return carry
  lax.fori_loop(0, ZROWS, _zero_row, 0)

  r0 = s * ROWS_PER_TILE
  for q in range(ROWS_PER_TILE // ZROWS):
    pltpu.sync_copy(zbuf, a_sh.at[pl.ds(r0 + q * ZROWS, ZROWS)])
    pltpu.sync_copy(zbuf_e, ec_sh.at[pl.ds(r0 + q * ZROWS, ZROWS)])
  plsc.subcore_barrier()

  tile_base = s * EDGES_PER_TILE

  def _run(x_hbm, use_ef):
    def _issue_loads(k, base):
      pltpu.async_copy(ei_hbm.at[0, pl.ds(base, CHUNK)], src[k], sem_ld[k])
      pltpu.async_copy(ei_hbm.at[1, pl.ds(base, CHUNK)], dst[k], sem_ld[k])
      if use_ef:
        pltpu.async_copy(ef_hbm.at[pl.ds(base, CHUNK), :], efv[k], sem_ld[k])

    def _wait_loads(k):
      pltpu.make_async_copy(ei_hbm.at[0, pl.ds(0, CHUNK)], src[k],
                            sem_ld[k]).wait()
      pltpu.make_async_copy(ei_hbm.at[1, pl.ds(0, CHUNK)], dst[k],
                            sem_ld[k]).wait()
      if use_ef:
        pltpu.make_async_copy(ef_hbm.at[pl.ds(0, CHUNK), :], efv[k],
                              sem_ld[k]).wait()

    def _wait_gather(k):
      pltpu.make_async_copy(x_hbm.at[src[k]], rows[k], sem_g[k]).wait()

    ec_rows = efv if use_ef else [ones_v] * NSLOT

    def _wait_scat(k):
      pltpu.make_async_copy(rows[k], a_sh.at[dst[k]], sem_s[k]).wait()
      pltpu.make_async_copy(ec_rows[k], ec_sh.at[dst[k]], sem_s[k]).wait()

    for k in range(NSLOT):
      _issue_loads(k, tile_base + k * CHUNK)

    def _group(g, carry):
      base0 = tile_base + g * (NSLOT * CHUNK)
      for k in range(NSLOT):
        _wait_loads(k)
        pltpu.async_copy(x_hbm.at[src[k]], rows[k], sem_g[k])
      for k in range(NSLOT):
        _wait_gather(k)
        pltpu.async_copy(rows[k], a_sh.at[dst[k]], sem_s[k], add=True)
        pltpu.async_copy(ec_rows[k], ec_sh.at[dst[k]], sem_s[k], add=True)
      for k in range(NSLOT):
        _wait_scat(k)

        @pl.when(g < GROUPS - 1)
        def _():
          _issue_loads(k, base0 + (NSLOT + k) * CHUNK)
      return carry
    lax.fori_loop(0, GROUPS, _group, 0)

  @pl.when(c == 0)
  def _():
    _run(xh0_hbm, use_ef=True)

  @pl.when(c == 1)
  def _():
    _run(xh1_hbm, use_ef=False)

  plsc.subcore_barrier()

  @pl.when(jnp.logical_and(c == 0, s == 0))
  def _():
    pltpu.sync_copy(a_sh, a_out.at[0])

  @pl.when(jnp.logical_and(c == 1, s == 0))
  def _():
    pltpu.sync_copy(a_sh, a_out.at[1])

  @pl.when(jnp.logical_and(c == 0, s == 1))
  def _():
    pltpu.sync_copy(ec_sh, e_out)

  @pl.when(jnp.logical_and(c == 1, s == 1))
  def _():
    pltpu.sync_copy(ec_sh, c_out)


_sc_call = pl.kernel(
    _sc_body,
    out_type=(
        jax.ShapeDtypeStruct((NC, N_NODES, D_HALF), jnp.float32),
        jax.ShapeDtypeStruct((N_NODES, D_EDGE), jnp.float32),
        jax.ShapeDtypeStruct((N_NODES, D_EDGE), jnp.float32),
    ),
    mesh=plsc.VectorSubcoreMesh(core_axis_name="c", subcore_axis_name="s"),
    compiler_params=pltpu.CompilerParams(use_tc_tiling_on_sc=False),
    scratch_types=(
        tuple(pltpu.VMEM((CHUNK,), jnp.int32) for _ in range(NSLOT))
        + tuple(pltpu.VMEM((CHUNK,), jnp.int32) for _ in range(NSLOT))
        + tuple(pltpu.VMEM((CHUNK, D_HALF), jnp.float32)
                for _ in range(NSLOT))
        + tuple(pltpu.VMEM((CHUNK, D_EDGE), jnp.float32)
                for _ in range(NSLOT))
        + (
            pltpu.VMEM((CHUNK, D_EDGE), jnp.float32),
            pltpu.VMEM((ZROWS, D_HALF), jnp.float32),
            pltpu.VMEM((ZROWS, D_EDGE), jnp.float32),
            pltpu.VMEM_SHARED((N_NODES, D_HALF), jnp.float32),
            pltpu.VMEM_SHARED((N_NODES, D_EDGE), jnp.float32),
        )
        + tuple(pltpu.SemaphoreType.DMA for _ in range(3 * NSLOT))
    ),
)


BLK = 1000


def _tc_body(a_ref, e_ref, c_ref, x_ref, ws_ref, wd_ref, we_ref, b_ref,
             o_ref):
  cnt = c_ref[:, 0:1]
  acc = jnp.dot(a_ref[0], ws_ref[:D_HALF, :],
                preferred_element_type=jnp.float32)
  acc += jnp.dot(a_ref[1], ws_ref[D_HALF:, :],
                 preferred_element_type=jnp.float32)
  acc += jnp.dot(e_ref[...], we_ref[...], preferred_element_type=jnp.float32)
  acc += cnt * (jnp.dot(x_ref[...], wd_ref[...],
                        preferred_element_type=jnp.float32) + b_ref[...])
  o_ref[...] = jnp.maximum(acc / jnp.maximum(cnt, 1.0), 0.0)


_tc_call = pl.pallas_call(
    _tc_body,
    grid=(N_NODES // BLK,),
    in_specs=[
        pl.BlockSpec((NC, BLK, D_HALF), lambda i: (0, i, 0)),
        pl.BlockSpec((BLK, D_EDGE), lambda i: (i, 0)),
        pl.BlockSpec((BLK, D_EDGE), lambda i: (i, 0)),
        pl.BlockSpec((BLK, D_NODE), lambda i: (i, 0)),
        pl.BlockSpec((D_NODE, D_OUT), lambda i: (0, 0)),
        pl.BlockSpec((D_NODE, D_OUT), lambda i: (0, 0)),
        pl.BlockSpec((D_EDGE, D_OUT), lambda i: (0, 0)),
        pl.BlockSpec((1, D_OUT), lambda i: (0, 0)),
    ],
    out_specs=pl.BlockSpec((BLK, D_OUT), lambda i: (i, 0)),
    out_shape=jax.ShapeDtypeStruct((N_NODES, D_OUT), jnp.float32),
)


def kernel(node_feats, edge_feats, edge_index, W, b):
  ei = edge_index.astype(jnp.int32)
  xh0 = node_feats[:, :D_HALF]
  xh1 = node_feats[:, D_HALF:]
  a_p, e_p, c_p = _sc_call(xh0, xh1, ei, edge_feats)
  ws_t = W[:, :D_NODE].T
  wd_t = W[:, D_NODE:2 * D_NODE].T
  we_t = W[:, 2 * D_NODE:].T
  return _tc_call(a_p, e_p, c_p, node_feats, ws_t, wd_t, we_t,
                  b.reshape(1, D_OUT))
